# baseline (speedup 1.0000x reference)
import functools

import jax
import jax.numpy as jnp
from jax import lax
from jax.experimental import pallas as pl
from jax.experimental.pallas import tpu as pltpu
from jax.experimental.pallas import tpu_sc as plsc

B = 16384
NCOV = 256

_info = plsc.get_sparse_core_info()
_NC = _info.num_cores
_NW = _info.num_cores * _info.num_subcores
_BPW = B // _NW
_ENT_PAD = 102400
_EPT = _ENT_PAD // 16


def _fe_gather(entity_fe, time_fe, entity_ids, time_ids):
    mesh = plsc.VectorSubcoreMesh(core_axis_name="c", subcore_axis_name="s")

    @functools.partial(
        pl.kernel,
        mesh=mesh,
        out_type=jax.ShapeDtypeStruct((B,), jnp.float32),
        scratch_types=[
            pltpu.VMEM((_BPW,), jnp.int32),
            pltpu.VMEM((_BPW,), jnp.int32),
            pltpu.VMEM((_BPW,), jnp.float32),
            pltpu.VMEM((_BPW,), jnp.float32),
            pltpu.VMEM_SHARED((_ENT_PAD,), jnp.float32),
            pltpu.VMEM_SHARED((256,), jnp.float32),
            pltpu.SemaphoreType.DMA,
            pltpu.SemaphoreType.DMA,
        ],
    )
    def k(ent_hbm, tim_hbm, eid_hbm, tid_hbm, out_hbm,
          eid_v, tid_v, ent_v, tim_v, ent_s, tim_s, sem_e, sem_t):
        cid = lax.axis_index("c")
        sid = lax.axis_index("s")
        wid = sid * _NC + cid
        base = wid * _BPW
        pltpu.sync_copy(ent_hbm.at[pl.ds(sid * _EPT, _EPT)],
                        ent_s.at[pl.ds(sid * _EPT, _EPT)])

        @pl.when(sid == 0)
        def _():
            pltpu.sync_copy(tim_hbm, tim_s)

        pltpu.sync_copy(eid_hbm.at[pl.ds(base, _BPW)], eid_v)
        pltpu.sync_copy(tid_hbm.at[pl.ds(base, _BPW)], tid_v)
        plsc.subcore_barrier()
        cp_e = pltpu.async_copy(ent_s.at[eid_v], ent_v, sem_e)
        cp_t = pltpu.async_copy(tim_s.at[tid_v], tim_v, sem_t)
        cp_e.wait()
        cp_t.wait()
        for i in range(_BPW // 16):
            sl = pl.ds(i * 16, 16)
            ent_v[sl] = ent_v[sl] + tim_v[sl]
        pltpu.sync_copy(ent_v, out_hbm.at[pl.ds(base, _BPW)])

    return k(entity_fe, time_fe, entity_ids, time_ids)


def _matvec(X, beta_row):
    BLK = 2048

    def body(x_ref, b_ref, o_ref):
        o_ref[...] = jax.lax.dot_general(
            b_ref[...], x_ref[...],
            (((1,), (1,)), ((), ())),
            preferred_element_type=jnp.float32,
        )[0]

    return pl.pallas_call(
        body,
        grid=(B // BLK,),
        in_specs=[
            pl.BlockSpec((BLK, NCOV), lambda i: (i, 0)),
            pl.BlockSpec((1, NCOV), lambda i: (0, 0)),
        ],
        out_specs=pl.BlockSpec((BLK,), lambda i: (i,)),
        out_shape=jax.ShapeDtypeStruct((B,), jnp.float32),
    )(X, beta_row)


def kernel(entity_ids, time_ids, X, entity_fe, time_fe, beta_w):
    eids = entity_ids.astype(jnp.int32)
    tids = time_ids.astype(jnp.int32)
    tim_pad = jnp.pad(time_fe.reshape(-1), (0, 256 - time_fe.shape[0]))
    ent_pad = jnp.pad(entity_fe.reshape(-1), (0, _ENT_PAD - entity_fe.shape[0]))
    fe = _fe_gather(ent_pad, tim_pad, eids, tids)
    pred = _matvec(X, beta_w)
    return pred + fe


# speedup vs baseline: 1.0322x; 1.0322x over previous
import functools

import jax
import jax.numpy as jnp
from jax import lax
from jax.experimental import pallas as pl
from jax.experimental.pallas import tpu as pltpu
from jax.experimental.pallas import tpu_sc as plsc

B = 16384
NCOV = 256

_info = plsc.get_sparse_core_info()
_NC = _info.num_cores
_NW = _info.num_cores * _info.num_subcores
_BPW = B // _NW
_ENT_PAD = 102400
_EPT = _ENT_PAD // 16


def _fe_gather(entity_fe, time_fe, entity_ids, time_ids):
    mesh = plsc.VectorSubcoreMesh(core_axis_name="c", subcore_axis_name="s")

    @functools.partial(
        pl.kernel,
        mesh=mesh,
        out_type=jax.ShapeDtypeStruct((B,), jnp.float32),
        scratch_types=[
            pltpu.VMEM((_BPW,), jnp.int32),
            pltpu.VMEM((_BPW,), jnp.int32),
            pltpu.VMEM((_BPW,), jnp.float32),
            pltpu.VMEM((_BPW,), jnp.float32),
            pltpu.VMEM_SHARED((_ENT_PAD,), jnp.float32),
            pltpu.VMEM_SHARED((256,), jnp.float32),
            pltpu.SemaphoreType.DMA,
            pltpu.SemaphoreType.DMA,
        ],
    )
    def k(ent_hbm, tim_hbm, eid_hbm, tid_hbm, out_hbm,
          eid_v, tid_v, ent_v, tim_v, ent_s, tim_s, sem_e, sem_t):
        cid = lax.axis_index("c")
        sid = lax.axis_index("s")
        wid = sid * _NC + cid
        base = wid * _BPW
        pltpu.sync_copy(ent_hbm.at[pl.ds(sid * _EPT, _EPT)],
                        ent_s.at[pl.ds(sid * _EPT, _EPT)])

        @pl.when(sid == 0)
        def _():
            pltpu.sync_copy(tim_hbm, tim_s)

        pltpu.sync_copy(eid_hbm.at[pl.ds(base, _BPW)], eid_v)
        pltpu.sync_copy(tid_hbm.at[pl.ds(base, _BPW)], tid_v)
        plsc.subcore_barrier()
        cp_e = pltpu.async_copy(ent_s.at[eid_v], ent_v, sem_e)
        cp_t = pltpu.async_copy(tim_s.at[tid_v], tim_v, sem_t)
        cp_e.wait()
        cp_t.wait()
        for i in range(_BPW // 16):
            sl = pl.ds(i * 16, 16)
            ent_v[sl] = ent_v[sl] + tim_v[sl]
        pltpu.sync_copy(ent_v, out_hbm.at[pl.ds(base, _BPW)])

    return k(entity_fe, time_fe, entity_ids, time_ids)


def _matvec(X, beta_row):
    BLK = 16384

    def body(x_ref, b_ref, o_ref):
        o_ref[...] = jax.lax.dot_general(
            b_ref[...], x_ref[...],
            (((1,), (1,)), ((), ())),
            preferred_element_type=jnp.float32,
        )[0]

    return pl.pallas_call(
        body,
        grid=(B // BLK,),
        in_specs=[
            pl.BlockSpec((BLK, NCOV), lambda i: (i, 0)),
            pl.BlockSpec((1, NCOV), lambda i: (0, 0)),
        ],
        out_specs=pl.BlockSpec((BLK,), lambda i: (i,)),
        out_shape=jax.ShapeDtypeStruct((B,), jnp.float32),
    )(X, beta_row)


def kernel(entity_ids, time_ids, X, entity_fe, time_fe, beta_w):
    eids = entity_ids.astype(jnp.int32)
    tids = time_ids.astype(jnp.int32)
    tim_pad = jnp.pad(time_fe.reshape(-1), (0, 256 - time_fe.shape[0]))
    ent_pad = jnp.pad(entity_fe.reshape(-1), (0, _ENT_PAD - entity_fe.shape[0]))
    fe = _fe_gather(ent_pad, tim_pad, eids, tids)
    pred = _matvec(X, beta_w)
    return pred + fe


# speedup vs baseline: 1.0704x; 1.0369x over previous
import functools

import jax
import jax.numpy as jnp
from jax import lax
from jax.experimental import pallas as pl
from jax.experimental.pallas import tpu as pltpu
from jax.experimental.pallas import tpu_sc as plsc

B = 16384
NCOV = 256

_info = plsc.get_sparse_core_info()
_NC = _info.num_cores
_NW = _info.num_cores * _info.num_subcores
_BPW = B // _NW
_ENT_PAD = 102400
_EPT = _ENT_PAD // 16


def _fe_gather(entity_fe, time_fe, entity_ids, time_ids):
    mesh = plsc.VectorSubcoreMesh(core_axis_name="c", subcore_axis_name="s")

    @functools.partial(
        pl.kernel,
        mesh=mesh,
        out_type=jax.ShapeDtypeStruct((B,), jnp.float32),
        scratch_types=[
            pltpu.VMEM((_BPW,), jnp.int32),
            pltpu.VMEM((_BPW,), jnp.int32),
            pltpu.VMEM((_BPW,), jnp.float32),
            pltpu.VMEM((_BPW,), jnp.float32),
            pltpu.VMEM_SHARED((_ENT_PAD,), jnp.float32),
            pltpu.VMEM_SHARED((256,), jnp.float32),
            pltpu.SemaphoreType.DMA,
            pltpu.SemaphoreType.DMA,
        ],
    )
    def k(ent_hbm, tim_hbm, eid_hbm, tid_hbm, out_hbm,
          eid_v, tid_v, ent_v, tim_v, ent_s, tim_s, sem_e, sem_t):
        cid = lax.axis_index("c")
        sid = lax.axis_index("s")
        wid = sid * _NC + cid
        base = wid * _BPW
        pltpu.sync_copy(ent_hbm.at[pl.ds(sid * _EPT, _EPT)],
                        ent_s.at[pl.ds(sid * _EPT, _EPT)])

        @pl.when(sid == 0)
        def _():
            pltpu.sync_copy(tim_hbm, tim_s)

        pltpu.sync_copy(eid_hbm.at[pl.ds(base, _BPW)], eid_v)
        pltpu.sync_copy(tid_hbm.at[pl.ds(base, _BPW)], tid_v)
        plsc.subcore_barrier()
        cp_e = pltpu.async_copy(ent_s.at[eid_v], ent_v, sem_e)
        cp_t = pltpu.async_copy(tim_s.at[tid_v], tim_v, sem_t)
        cp_e.wait()
        cp_t.wait()
        for i in range(_BPW // 16):
            sl = pl.ds(i * 16, 16)
            ent_v[sl] = ent_v[sl] + tim_v[sl]
        pltpu.sync_copy(ent_v, out_hbm.at[pl.ds(base, _BPW)])

    return k(entity_fe, time_fe, entity_ids, time_ids)


def _matvec(X, beta_row):
    BLK = 8192

    def body(x_ref, b_ref, o_ref):
        o_ref[...] = jax.lax.dot_general(
            b_ref[...], x_ref[...],
            (((1,), (1,)), ((), ())),
            preferred_element_type=jnp.float32,
        )[0]

    return pl.pallas_call(
        body,
        grid=(B // BLK,),
        in_specs=[
            pl.BlockSpec((BLK, NCOV), lambda i: (i, 0)),
            pl.BlockSpec((1, NCOV), lambda i: (0, 0)),
        ],
        out_specs=pl.BlockSpec((BLK,), lambda i: (i,)),
        out_shape=jax.ShapeDtypeStruct((B,), jnp.float32),
    )(X, beta_row)


def kernel(entity_ids, time_ids, X, entity_fe, time_fe, beta_w):
    eids = entity_ids.astype(jnp.int32)
    tids = time_ids.astype(jnp.int32)
    tim_pad = jnp.pad(time_fe.reshape(-1), (0, 256 - time_fe.shape[0]))
    ent_pad = jnp.pad(entity_fe.reshape(-1), (0, _ENT_PAD - entity_fe.shape[0]))
    fe = _fe_gather(ent_pad, tim_pad, eids, tids)
    pred = _matvec(X, beta_w)
    return pred + fe
